# baseline (speedup 1.0000x reference)
import functools

import jax
import jax.numpy as jnp
from jax import lax
from jax.experimental import pallas as pl
from jax.experimental.pallas import tpu as pltpu
from jax.experimental.pallas import tpu_sc as plsc

_F32 = jnp.float32
_CH = 128


def _kron3(W):
    return jnp.kron(jnp.eye(3, dtype=W.dtype), W)


def _sc_gather(table, idx2d, n_rows):
    D = table.shape[1]
    info = plsc.get_sparse_core_info()
    nw = info.num_cores * info.num_subcores
    total_chunks = idx2d.shape[0]
    chunks_pw = total_chunks // nw
    mesh = plsc.VectorSubcoreMesh(core_axis_name="c", subcore_axis_name="s")

    @functools.partial(
        pl.kernel,
        mesh=mesh,
        out_type=jax.ShapeDtypeStruct((n_rows, D), _F32),
        scratch_types=[
            pltpu.VMEM((chunks_pw, _CH), jnp.int32),
            pltpu.VMEM((_CH, D), _F32),
            pltpu.VMEM((_CH, D), _F32),
            pltpu.SemaphoreType.DMA,
            pltpu.SemaphoreType.DMA,
        ],
    )
    def k(table_hbm, idx_hbm, out_hbm, idx_v, rows0, rows1, sem0, sem1):
        wid = lax.axis_index("s") * info.num_cores + lax.axis_index("c")
        pltpu.sync_copy(idx_hbm.at[pl.ds(wid * chunks_pw, chunks_pw)], idx_v)
        base = wid * (chunks_pw * _CH)

        pltpu.async_copy(table_hbm.at[idx_v.at[0]], rows0, sem0)

        def body(tt, _):
            t0 = 2 * tt

            @pl.when(t0 + 1 < chunks_pw)
            def _():
                pltpu.async_copy(table_hbm.at[idx_v.at[t0 + 1]], rows1, sem1)

            pltpu.make_async_copy(table_hbm.at[idx_v.at[t0]], rows0, sem0).wait()
            pltpu.sync_copy(rows0, out_hbm.at[pl.ds(base + t0 * _CH, _CH)])

            @pl.when(t0 + 1 < chunks_pw)
            def _():
                @pl.when(t0 + 2 < chunks_pw)
                def _():
                    pltpu.async_copy(table_hbm.at[idx_v.at[t0 + 2]], rows0, sem0)

                pltpu.make_async_copy(
                    table_hbm.at[idx_v.at[t0 + 1]], rows1, sem1
                ).wait()
                pltpu.sync_copy(
                    rows1, out_hbm.at[pl.ds(base + (t0 + 1) * _CH, _CH)]
                )

            return 0

        lax.fori_loop(0, (chunks_pw + 1) // 2, body, 0)

    return k(table, idx2d)


def _dt(a, b):
    return lax.dot_general(
        a, b, (((1,), (0,)), ((), ())),
        precision=lax.Precision.DEFAULT, preferred_element_type=_F32,
    )


def _tile3(g):
    return jnp.concatenate([g, g, g], axis=-1)


def _vsq(Vh, h):
    V2 = Vh * Vh
    return V2[:, :h] + V2[:, h:2 * h] + V2[:, 2 * h:]


def _tc_body(*refs, blk, K):
    (G_ref, hE_ref, hV_ref, m_ref), w, out_ref = refs[:4], refs[4:-1], refs[-1]
    (Whn, Whe3, Wheg, Wsn, Wse, bs0, Wv0,
     Wh1, Ws1, b1, Wv1,
     Wh2, Ws2, b2, Wv2,
     g0, be0,
     Whd0, Wsd0, bd0, Wvd0,
     Whd1, Wsd1, bd1, Wvd1,
     g1, be1) = (r[...] for r in w)
    R = blk * K

    mask_n = m_ref[...]
    mask_e = jnp.broadcast_to(mask_n[:, None, :], (blk, K, 1)).reshape(R, 1)
    G = G_ref[...] * mask_e
    hE = hE_ref[...] * mask_e
    hV = hV_ref[...]

    nodeV = _dt(hV[:, :48], Whn)
    nodeS = _dt(hV[:, 48:], Wsn)
    Vh = _dt(G[:, 120:168], Wheg) + _dt(hE[:, :3], Whe3)
    Vh = (Vh.reshape(blk, K, 99) + nodeV[:, None, :]).reshape(R, 99)
    vn = jnp.sqrt(_vsq(Vh, 33) + 1e-8)
    s_edge = jnp.concatenate([hE[:, 3:35], G[:, :120], vn], axis=-1)
    s0 = _dt(s_edge, Wse)
    s0 = (s0.reshape(blk, K, 100) + nodeS[:, None, :]).reshape(R, 100) + bs0
    s0 = jnp.maximum(s0, 0.0)
    Vo = _dt(Vh, Wv0)
    Vo = Vo * _tile3(jax.nn.sigmoid(jnp.sqrt(_vsq(Vo, 16) + 1e-8)))

    Vh1 = _dt(Vo, Wh1)
    vn1 = jnp.sqrt(_vsq(Vh1, 16) + 1e-8)
    s1 = jnp.maximum(_dt(jnp.concatenate([s0, vn1], axis=-1), Ws1) + b1, 0.0)
    Vo1 = _dt(Vh1, Wv1)
    Vo1 = Vo1 * _tile3(jax.nn.sigmoid(jnp.sqrt(_vsq(Vo1, 16) + 1e-8)))

    Vh2 = _dt(Vo1, Wh2)
    vn2 = jnp.sqrt(_vsq(Vh2, 16) + 1e-8)
    s2 = _dt(jnp.concatenate([s1, vn2], axis=-1), Ws2) + b2
    Vo2 = _dt(Vh2, Wv2)

    Vm = jnp.mean(Vo2.reshape(blk, K, 48), axis=1)
    sm = jnp.mean(s2.reshape(blk, K, 100), axis=1)

    Vm = Vm / jnp.sqrt(
        jnp.sum(Vm * Vm, axis=-1, keepdims=True) / 16.0 + 1e-8)
    mu = jnp.mean(sm, axis=-1, keepdims=True)
    xc = sm - mu
    var = jnp.mean(xc * xc, axis=-1, keepdims=True)
    sm = xc / jnp.sqrt(var + 1e-5) * g0 + be0

    Vhd = _dt(Vm, Whd0)
    vnd = jnp.sqrt(_vsq(Vhd, 32) + 1e-8)
    sd = jnp.maximum(
        _dt(jnp.concatenate([sm, vnd], axis=-1), Wsd0) + bd0, 0.0)
    Vod = _dt(Vhd, Wvd0)
    Vod = Vod * _tile3(jax.nn.sigmoid(jnp.sqrt(_vsq(Vod, 32) + 1e-8)))

    Vhd1 = _dt(Vod, Whd1)
    vnd1 = jnp.sqrt(_vsq(Vhd1, 32) + 1e-8)
    sd1 = _dt(jnp.concatenate([sd, vnd1], axis=-1), Wsd1) + bd1
    Vod1 = _dt(Vhd1, Wvd1)

    Vf = Vod1 / jnp.sqrt(
        jnp.sum(Vod1 * Vod1, axis=-1, keepdims=True) / 16.0 + 1e-8)
    mu1 = jnp.mean(sd1, axis=-1, keepdims=True)
    xc1 = sd1 - mu1
    var1 = jnp.mean(xc1 * xc1, axis=-1, keepdims=True)
    sf = xc1 / jnp.sqrt(var1 + 1e-5) * g1 + be1

    out_ref[...] = jnp.concatenate([Vf, sf], axis=-1) * mask_n


def _tc_layer(G, hE, hV, mask_c, warrs, blk):
    BN = hV.shape[0]
    K = G.shape[0] // BN
    R = blk * K

    def wspec(a):
        nd = a.ndim
        return pl.BlockSpec(a.shape, lambda i, _n=nd: (0,) * _n)

    in_specs = [
        pl.BlockSpec((R, G.shape[1]), lambda i: (i, 0)),
        pl.BlockSpec((R, hE.shape[1]), lambda i: (i, 0)),
        pl.BlockSpec((blk, hV.shape[1]), lambda i: (i, 0)),
        pl.BlockSpec((blk, 1), lambda i: (i, 0)),
    ] + [wspec(a) for a in warrs]
    return pl.pallas_call(
        functools.partial(_tc_body, blk=blk, K=K),
        grid=(BN // blk,),
        in_specs=in_specs,
        out_specs=pl.BlockSpec((blk, 148), lambda i: (i, 0)),
        out_shape=jax.ShapeDtypeStruct((BN, 148), _F32),
    )(G, hE, hV, mask_c, *warrs)


def _prep_layer(p):
    k3 = _kron3
    Wh0 = p["wev0"]["Wh"]
    return [
        k3(Wh0[0:16]), k3(Wh0[16:17]), k3(Wh0[17:33]),
        p["wev0"]["Ws"][:100], p["wev0"]["Ws"][100:],
        p["wev0"]["bs"][None, :], k3(p["wev0"]["Wv"]),
        k3(p["wev1"]["Wh"]), p["wev1"]["Ws"],
        p["wev1"]["bs"][None, :], k3(p["wev1"]["Wv"]),
        k3(p["wev2"]["Wh"]), p["wev2"]["Ws"],
        p["wev2"]["bs"][None, :], k3(p["wev2"]["Wv"]),
        p["norm0"]["gamma"][None, :], p["norm0"]["beta"][None, :],
        k3(p["wdh0"]["Wh"]), p["wdh0"]["Ws"],
        p["wdh0"]["bs"][None, :], k3(p["wdh0"]["Wv"]),
        k3(p["wdh1"]["Wh"]), p["wdh1"]["Ws"],
        p["wdh1"]["bs"][None, :], k3(p["wdh1"]["Wv"]),
        p["norm1"]["gamma"][None, :], p["norm1"]["beta"][None, :],
    ]


def _to_cb(hV):
    v = hV[..., :48].reshape(hV.shape[:-1] + (16, 3))
    v = jnp.swapaxes(v, -1, -2).reshape(hV.shape[:-1] + (48,))
    return jnp.concatenate([v, hV[..., 48:]], axis=-1)


def _from_cb(hV):
    v = hV[..., :48].reshape(hV.shape[:-1] + (3, 16))
    v = jnp.swapaxes(v, -1, -2).reshape(hV.shape[:-1] + (48,))
    return jnp.concatenate([v, hV[..., 48:]], axis=-1)


def kernel(h_V, h_S, h_E, E_idx, mask, params):
    B, N, K = E_idx.shape
    BN, BNK = B * N, B * N * K
    D = 256

    h_V = _to_cb(h_V.astype(_F32))
    h_S = h_S.astype(_F32)
    h_E = h_E.astype(_F32)
    mask = mask.astype(_F32)

    ii = jnp.arange(N, dtype=jnp.int32)[None, :, None]
    ei = E_idx.astype(jnp.int32)
    attend = ei < ii
    cidx = ei + jnp.where(attend, 0, N) \
        + (2 * N) * jnp.arange(B, dtype=jnp.int32)[:, None, None]
    idx2d = cidx.reshape(-1, _CH)

    pad = jnp.zeros((B, N, D - 168), _F32)
    zS = jnp.zeros_like(h_S)
    half1 = jnp.concatenate([zS, h_V[..., 48:], h_V[..., :48], pad], axis=-1)

    hE_flat = h_E.reshape(BNK, h_E.shape[-1])
    mask_c = mask.reshape(BN, 1)

    prepped = [_prep_layer(p) for p in params]

    hV_cur = h_V
    for wl in prepped:
        half0 = jnp.concatenate(
            [h_S, hV_cur[..., 48:], hV_cur[..., :48], pad], axis=-1)
        table = jnp.concatenate([half0, half1], axis=1).reshape(2 * BN, D)
        G = _sc_gather(table, idx2d, BNK)
        hVn = _tc_layer(G, hE_flat, hV_cur.reshape(BN, 148), mask_c,
                        wl, blk=256)
        hV_cur = hVn.reshape(B, N, 148)
    return _from_cb(hV_cur)


# speedup vs baseline: 1.5051x; 1.5051x over previous
import functools

import jax
import jax.numpy as jnp
from jax import lax
from jax.experimental import pallas as pl
from jax.experimental.pallas import tpu as pltpu
from jax.experimental.pallas import tpu_sc as plsc

_F32 = jnp.float32
_CH = 128


def _expand3(W):
    vi, h = W.shape
    eye3 = jnp.eye(3, dtype=W.dtype)
    return (W[:, None, :, None] * eye3[None, :, None, :]).reshape(vi * 3, h * 3)


def _sel(h):
    return jnp.repeat(jnp.eye(h, dtype=_F32), 3, axis=0)


def _gateM(h):
    return _sel(h) @ _sel(h).T


def _sc_gather(table, idx2d, n_rows):
    D = table.shape[1]
    info = plsc.get_sparse_core_info()
    nw = info.num_cores * info.num_subcores
    total_chunks = idx2d.shape[0]
    chunks_pw = total_chunks // nw
    mesh = plsc.VectorSubcoreMesh(core_axis_name="c", subcore_axis_name="s")

    @functools.partial(
        pl.kernel,
        mesh=mesh,
        out_type=jax.ShapeDtypeStruct((n_rows, D), _F32),
        scratch_types=[
            pltpu.VMEM((chunks_pw, _CH), jnp.int32),
            pltpu.VMEM((_CH, D), _F32),
            pltpu.VMEM((_CH, D), _F32),
            pltpu.SemaphoreType.DMA,
            pltpu.SemaphoreType.DMA,
        ],
    )
    def k(table_hbm, idx_hbm, out_hbm, idx_v, rows0, rows1, sem0, sem1):
        wid = lax.axis_index("s") * info.num_cores + lax.axis_index("c")
        pltpu.sync_copy(idx_hbm.at[pl.ds(wid * chunks_pw, chunks_pw)], idx_v)
        base = wid * (chunks_pw * _CH)

        pltpu.async_copy(table_hbm.at[idx_v.at[0]], rows0, sem0)

        def body(tt, _):
            t0 = 2 * tt

            @pl.when(t0 + 1 < chunks_pw)
            def _():
                pltpu.async_copy(table_hbm.at[idx_v.at[t0 + 1]], rows1, sem1)

            pltpu.make_async_copy(table_hbm.at[idx_v.at[t0]], rows0, sem0).wait()
            pltpu.sync_copy(rows0, out_hbm.at[pl.ds(base + t0 * _CH, _CH)])

            @pl.when(t0 + 1 < chunks_pw)
            def _():
                @pl.when(t0 + 2 < chunks_pw)
                def _():
                    pltpu.async_copy(table_hbm.at[idx_v.at[t0 + 2]], rows0, sem0)

                pltpu.make_async_copy(
                    table_hbm.at[idx_v.at[t0 + 1]], rows1, sem1
                ).wait()
                pltpu.sync_copy(
                    rows1, out_hbm.at[pl.ds(base + (t0 + 1) * _CH, _CH)]
                )

            return 0

        lax.fori_loop(0, (chunks_pw + 1) // 2, body, 0)

    return k(table, idx2d)


def _dt(a, b):
    return lax.dot_general(
        a, b, (((1,), (0,)), ((), ())),
        precision=lax.Precision.DEFAULT, preferred_element_type=_F32,
    )


def _tc_body(*refs, blk, K):
    (G_ref, hE_ref, hV_ref, m_ref), w, out_ref = refs[:4], refs[4:-1], refs[-1]
    (Whn, Whe, Wsn, Wse, bs0, Wv0,
     Wh1, Ws1, b1, Wov1,
     Wh2, Ws2, b2, Wov2,
     g0, be0,
     Whd0, Wsd0, bd0, Wvd0,
     Whd1, Wsd1, bd1, Wvd1,
     g1, be1,
     S33, M16, M32, S32) = (r[...] for r in w)
    R = blk * K

    mask_n = m_ref[...]
    mask_e = jnp.broadcast_to(mask_n[:, None, :], (blk, K, 1)).reshape(R, 1)
    G = G_ref[...] * mask_e
    hE = hE_ref[...] * mask_e
    hV = hV_ref[...]

    nodeV = _dt(hV[:, :48], Whn)
    nodeS = _dt(hV[:, 48:], Wsn)
    v_edge = jnp.concatenate([hE[:, :3], G[:, 120:168]], axis=-1)
    Vh = _dt(v_edge, Whe)
    Vh = (Vh.reshape(blk, K, 99) + nodeV[:, None, :]).reshape(R, 99)
    vn = jnp.sqrt(_dt(Vh * Vh, S33) + 1e-8)
    s_edge = jnp.concatenate([hE[:, 3:35], G[:, :120], vn], axis=-1)
    s0 = _dt(s_edge, Wse)
    s0 = (s0.reshape(blk, K, 100) + nodeS[:, None, :]).reshape(R, 100) + bs0
    s0 = jnp.maximum(s0, 0.0)
    Vo = _dt(Vh, Wv0)
    Vo = Vo * jax.nn.sigmoid(jnp.sqrt(_dt(Vo * Vo, M16) + 1e-8))

    Vh1 = _dt(Vo, Wh1)
    x1 = _dt(jnp.concatenate([Vh1, Vh1 * Vh1], axis=-1), Wov1)
    Vo1 = x1[:, :48]
    vn1 = jnp.sqrt(x1[:, 48:] + 1e-8)
    s1 = jnp.maximum(_dt(jnp.concatenate([s0, vn1], axis=-1), Ws1) + b1, 0.0)
    Vo1 = Vo1 * jax.nn.sigmoid(jnp.sqrt(_dt(Vo1 * Vo1, M16) + 1e-8))

    Vh2 = _dt(Vo1, Wh2)
    x2 = _dt(jnp.concatenate([Vh2, Vh2 * Vh2], axis=-1), Wov2)
    Vo2 = x2[:, :48]
    vn2 = jnp.sqrt(x2[:, 48:] + 1e-8)
    s2 = _dt(jnp.concatenate([s1, vn2], axis=-1), Ws2) + b2

    Vm = jnp.mean(Vo2.reshape(blk, K, 48), axis=1)
    sm = jnp.mean(s2.reshape(blk, K, 100), axis=1)

    Vm = Vm / jnp.sqrt(
        jnp.sum(Vm * Vm, axis=-1, keepdims=True) / 16.0 + 1e-8)
    mu = jnp.mean(sm, axis=-1, keepdims=True)
    xc = sm - mu
    var = jnp.mean(xc * xc, axis=-1, keepdims=True)
    sm = xc / jnp.sqrt(var + 1e-5) * g0 + be0

    Vhd = _dt(Vm, Whd0)
    vnd = jnp.sqrt(_dt(Vhd * Vhd, S32) + 1e-8)
    sd = jnp.maximum(
        _dt(jnp.concatenate([sm, vnd], axis=-1), Wsd0) + bd0, 0.0)
    Vod = _dt(Vhd, Wvd0)
    Vod = Vod * jax.nn.sigmoid(jnp.sqrt(_dt(Vod * Vod, M32) + 1e-8))

    Vhd1 = _dt(Vod, Whd1)
    vnd1 = jnp.sqrt(_dt(Vhd1 * Vhd1, S32) + 1e-8)
    sd1 = _dt(jnp.concatenate([sd, vnd1], axis=-1), Wsd1) + bd1
    Vod1 = _dt(Vhd1, Wvd1)

    Vf = Vod1 / jnp.sqrt(
        jnp.sum(Vod1 * Vod1, axis=-1, keepdims=True) / 16.0 + 1e-8)
    mu1 = jnp.mean(sd1, axis=-1, keepdims=True)
    xc1 = sd1 - mu1
    var1 = jnp.mean(xc1 * xc1, axis=-1, keepdims=True)
    sf = xc1 / jnp.sqrt(var1 + 1e-5) * g1 + be1

    out_ref[...] = jnp.concatenate([Vf, sf], axis=-1) * mask_n


def _tc_layer(G, hE, hV, mask_c, warrs, blk):
    BN = hV.shape[0]
    K = G.shape[0] // BN
    R = blk * K

    def wspec(a):
        nd = a.ndim
        return pl.BlockSpec(a.shape, lambda i, _n=nd: (0,) * _n)

    in_specs = [
        pl.BlockSpec((R, G.shape[1]), lambda i: (i, 0)),
        pl.BlockSpec((R, hE.shape[1]), lambda i: (i, 0)),
        pl.BlockSpec((blk, hV.shape[1]), lambda i: (i, 0)),
        pl.BlockSpec((blk, 1), lambda i: (i, 0)),
    ] + [wspec(a) for a in warrs]
    return pl.pallas_call(
        functools.partial(_tc_body, blk=blk, K=K),
        grid=(BN // blk,),
        in_specs=in_specs,
        out_specs=pl.BlockSpec((blk, 148), lambda i: (i, 0)),
        out_shape=jax.ShapeDtypeStruct((BN, 148), _F32),
    )(G, hE, hV, mask_c, *warrs)


def _prep_layer(p):
    e3 = _expand3
    Wh0 = e3(p["wev0"]["Wh"])
    z48_16 = jnp.zeros((48, 16), _F32)
    Wov1 = jnp.concatenate([
        jnp.concatenate([e3(p["wev1"]["Wv"]), z48_16], axis=1),
        jnp.concatenate([jnp.zeros((48, 48), _F32), _sel(16)], axis=1),
    ], axis=0)
    Wov2 = jnp.concatenate([
        jnp.concatenate([e3(p["wev2"]["Wv"]), z48_16], axis=1),
        jnp.concatenate([jnp.zeros((48, 48), _F32), _sel(16)], axis=1),
    ], axis=0)
    return [
        Wh0[:48], Wh0[48:],
        p["wev0"]["Ws"][:100], p["wev0"]["Ws"][100:],
        p["wev0"]["bs"][None, :], e3(p["wev0"]["Wv"]),
        e3(p["wev1"]["Wh"]), p["wev1"]["Ws"],
        p["wev1"]["bs"][None, :], Wov1,
        e3(p["wev2"]["Wh"]), p["wev2"]["Ws"],
        p["wev2"]["bs"][None, :], Wov2,
        p["norm0"]["gamma"][None, :], p["norm0"]["beta"][None, :],
        e3(p["wdh0"]["Wh"]), p["wdh0"]["Ws"],
        p["wdh0"]["bs"][None, :], e3(p["wdh0"]["Wv"]),
        e3(p["wdh1"]["Wh"]), p["wdh1"]["Ws"],
        p["wdh1"]["bs"][None, :], e3(p["wdh1"]["Wv"]),
        p["norm1"]["gamma"][None, :], p["norm1"]["beta"][None, :],
    ]


def kernel(h_V, h_S, h_E, E_idx, mask, params):
    B, N, K = E_idx.shape
    BN, BNK = B * N, B * N * K
    D = 256

    h_V = h_V.astype(_F32)
    h_S = h_S.astype(_F32)
    h_E = h_E.astype(_F32)
    mask = mask.astype(_F32)

    ii = jnp.arange(N, dtype=jnp.int32)[None, :, None]
    ei = E_idx.astype(jnp.int32)
    attend = ei < ii
    cidx = ei + jnp.where(attend, 0, N) \
        + (2 * N) * jnp.arange(B, dtype=jnp.int32)[:, None, None]
    idx2d = cidx.reshape(-1, _CH)

    pad = jnp.zeros((B, N, D - 168), _F32)
    zS = jnp.zeros_like(h_S)
    half1 = jnp.concatenate([zS, h_V[..., 48:], h_V[..., :48], pad], axis=-1)

    hE_flat = h_E.reshape(BNK, h_E.shape[-1])
    mask_c = mask.reshape(BN, 1)

    sels = [_sel(33), _gateM(16), _gateM(32), _sel(32)]
    prepped = [_prep_layer(p) for p in params]

    hV_cur = h_V
    for wl in prepped:
        half0 = jnp.concatenate(
            [h_S, hV_cur[..., 48:], hV_cur[..., :48], pad], axis=-1)
        table = jnp.concatenate([half0, half1], axis=1).reshape(2 * BN, D)
        G = _sc_gather(table, idx2d, BNK)
        hVn = _tc_layer(G, hE_flat, hV_cur.reshape(BN, 148), mask_c,
                        wl + sels, blk=256)
        hV_cur = hVn.reshape(B, N, 148)
    return hV_cur
